# baseline (speedup 1.0000x reference)
import functools

import numpy as np
import jax
import jax.numpy as jnp
from jax import lax
from jax.experimental import pallas as pl
from jax.experimental.pallas import tpu as pltpu
from jax.experimental.pallas import tpu_sc as plsc

_N_WORKERS = 32
_N_CORES = 2


def _linspace_floor_idx(t, t_out):
    i = np.arange(t_out - 1, dtype=np.float32)
    frac = i / np.float32(t_out - 1)
    vals = np.float32(0.0) * (np.float32(1.0) - frac) + np.float32(t - 1) * frac
    return np.concatenate([vals, [np.float32(t - 1)]]).astype(np.int32)


def _sc_frame_gather(frames, pairs, t_out):
    c, t, h, w = frames.shape
    n_planes = len(pairs)
    qs = 4
    n_chunks = qs * n_planes
    assert n_chunks % _N_WORKERS == 0
    per_w = n_chunks // _N_WORKERS
    qh = h // qs

    mesh = plsc.VectorSubcoreMesh(core_axis_name="c", subcore_axis_name="s")

    @functools.partial(
        pl.kernel,
        mesh=mesh,
        out_type=jax.ShapeDtypeStruct((c, t_out, h, w), frames.dtype),
        scratch_types=[
            pltpu.VMEM((per_w, qh, w), frames.dtype),
            pltpu.SemaphoreType.DMA,
            pltpu.SemaphoreType.DMA,
        ],
    )
    def k(frames_ref, out_ref, buf, sem_a, sem_b):
        wid = lax.axis_index("s") * _N_CORES + lax.axis_index("c")
        half = per_w // 2
        for wo in range(_N_WORKERS):

            @pl.when(wid == wo)
            def _():
                def src(k_):
                    q = wo * per_w + k_
                    j, qi = q // qs, q % qs
                    ci, ti = pairs[j]
                    return frames_ref.at[ci, ti, pl.ds(qi * qh, qh), :]

                def dst(k_):
                    q = wo * per_w + k_
                    j, qi = q // qs, q % qs
                    return out_ref.at[j // t_out, j % t_out, pl.ds(qi * qh, qh), :]

                r0 = [pltpu.async_copy(src(k_), buf.at[k_], sem_a)
                      for k_ in range(half)]
                r1 = [pltpu.async_copy(src(k_), buf.at[k_], sem_b)
                      for k_ in range(half, per_w)]
                for r in r0:
                    r.wait()
                w0 = [pltpu.async_copy(buf.at[k_], dst(k_), sem_a)
                      for k_ in range(half)]
                for r in r1:
                    r.wait()
                w1 = [pltpu.async_copy(buf.at[k_], dst(k_), sem_b)
                      for k_ in range(half, per_w)]
                for wr in w0:
                    wr.wait()
                for wr in w1:
                    wr.wait()

    return k(frames)


def _tc_copy(frames):
    c, t, h, w = frames.shape
    tb = 32

    def body(src, dst):
        dst[...] = src[...]

    return pl.pallas_call(
        body,
        grid=(c, t // tb),
        in_specs=[pl.BlockSpec((1, tb, h, w), lambda ci, ti: (ci, ti, 0, 0))],
        out_specs=pl.BlockSpec((1, tb, h, w), lambda ci, ti: (ci, ti, 0, 0)),
        out_shape=jax.ShapeDtypeStruct(frames.shape, frames.dtype),
    )(frames)


def kernel(frames):
    c, t, h, w = frames.shape
    t_out = t // 4
    idx = _linspace_floor_idx(t, t_out)
    pairs = tuple((ci, int(ti)) for ci in range(c) for ti in idx)
    slow = _sc_frame_gather(frames, pairs, t_out)
    fast = _tc_copy(frames)
    return slow, fast


# speedup vs baseline: 1.0029x; 1.0029x over previous
import functools

import numpy as np
import jax
from jax import lax
from jax.experimental import pallas as pl
from jax.experimental.pallas import tpu as pltpu
from jax.experimental.pallas import tpu_sc as plsc

_N_WORKERS = 32
_N_CORES = 2


def _linspace_floor_idx(t, t_out):
    i = np.arange(t_out - 1, dtype=np.float32)
    frac = i / np.float32(t_out - 1)
    vals = np.float32(0.0) * (np.float32(1.0) - frac) + np.float32(t - 1) * frac
    return np.concatenate([vals, [np.float32(t - 1)]]).astype(np.int32)


def _sc_frame_gather(frames, pairs, t_out):
    c, t, h, w = frames.shape
    n_planes = len(pairs)
    qs = 4
    n_chunks = qs * n_planes
    assert n_chunks % _N_WORKERS == 0
    per_w = n_chunks // _N_WORKERS
    qh = h // qs

    mesh = plsc.VectorSubcoreMesh(core_axis_name="c", subcore_axis_name="s")

    @functools.partial(
        pl.kernel,
        mesh=mesh,
        out_type=jax.ShapeDtypeStruct((c, t_out, h, w), frames.dtype),
        scratch_types=[
            pltpu.VMEM((per_w, qh, w), frames.dtype),
            pltpu.SemaphoreType.DMA,
            pltpu.SemaphoreType.DMA,
        ],
    )
    def k(frames_ref, out_ref, buf, sem_a, sem_b):
        wid = lax.axis_index("s") * _N_CORES + lax.axis_index("c")
        half = per_w // 2
        for wo in range(_N_WORKERS):

            @pl.when(wid == wo)
            def _():
                def src(k_):
                    q = wo * per_w + k_
                    j, qi = q // qs, q % qs
                    ci, ti = pairs[j]
                    return frames_ref.at[ci, ti, pl.ds(qi * qh, qh), :]

                def dst(k_):
                    q = wo * per_w + k_
                    j, qi = q // qs, q % qs
                    return out_ref.at[j // t_out, j % t_out, pl.ds(qi * qh, qh), :]

                r0 = [pltpu.async_copy(src(k_), buf.at[k_], sem_a)
                      for k_ in range(half)]
                r1 = [pltpu.async_copy(src(k_), buf.at[k_], sem_b)
                      for k_ in range(half, per_w)]
                for r in r0:
                    r.wait()
                w0 = [pltpu.async_copy(buf.at[k_], dst(k_), sem_a)
                      for k_ in range(half)]
                for r in r1:
                    r.wait()
                w1 = [pltpu.async_copy(buf.at[k_], dst(k_), sem_b)
                      for k_ in range(half, per_w)]
                for wr in w0:
                    wr.wait()
                for wr in w1:
                    wr.wait()

    return k(frames)


def _tc_copy(frames):
    c, t, h, w = frames.shape
    tb = 32

    def body(src, dst):
        dst[...] = src[...]

    return pl.pallas_call(
        body,
        grid=(c, t // tb),
        in_specs=[pl.BlockSpec((1, tb, h, w), lambda ci, ti: (ci, ti, 0, 0))],
        out_specs=pl.BlockSpec((1, tb, h, w), lambda ci, ti: (ci, ti, 0, 0)),
        out_shape=jax.ShapeDtypeStruct(frames.shape, frames.dtype),
    )(frames)


def kernel(frames):
    c, t, h, w = frames.shape
    t_out = t // 4
    idx = _linspace_floor_idx(t, t_out)
    pairs = tuple((ci, int(ti)) for ci in range(c) for ti in idx)
    slow = _sc_frame_gather(frames, pairs, t_out)
    fast = _tc_copy(frames)
    return slow, fast
